# Initial kernel scaffold; baseline (speedup 1.0000x reference)
import jax
import jax.numpy as jnp
from jax.experimental import pallas as pl


def kernel(inp, W):
    raise NotImplementedError("write your pallas kernel here")


# speedup vs baseline: 1.0334x; 1.0334x over previous
import functools

import jax
import jax.numpy as jnp
from jax import lax
from jax.experimental import pallas as pl
from jax.experimental.pallas import tpu as pltpu

N_STEPS_ = 8
N_OUT_ = 128


def _mm_body(x_ref, w_ref, o_ref, *, sigmoid):
    acc = lax.dot_general(
        x_ref[...], w_ref[...],
        dimension_numbers=(((1,), (1,)), ((), ())),
        preferred_element_type=jnp.float32,
    )
    o_ref[...] = jax.nn.sigmoid(acc) if sigmoid else acc


def _step(x, W, *, k_size, n_block_start, n_blocks, bn, sigmoid):
    B = x.shape[0]
    body = functools.partial(_mm_body, sigmoid=sigmoid)
    return pl.pallas_call(
        body,
        grid=(n_blocks,),
        in_specs=[
            pl.BlockSpec((B, k_size), lambda n: (0, 0)),
            pl.BlockSpec((bn, k_size), lambda n, _s=n_block_start: (n + _s, 0)),
        ],
        out_specs=pl.BlockSpec((B, bn), lambda n: (0, n)),
        out_shape=jax.ShapeDtypeStruct((B, n_blocks * bn), jnp.float32),
        compiler_params=pltpu.CompilerParams(
            vmem_limit_bytes=100 * 1024 * 1024,
        ),
    )(x, W)


def kernel(inp, W):
    B, n_inputs = inp.shape
    H = W.shape[0]
    bn = 256
    h = _step(inp, W, k_size=n_inputs, n_block_start=0,
              n_blocks=H // bn, bn=bn, sigmoid=True)
    for _ in range(N_STEPS_ - 2):
        h = _step(h, W, k_size=H, n_block_start=0,
                  n_blocks=H // bn, bn=bn, sigmoid=True)
    out = _step(h, W, k_size=H, n_block_start=(H - N_OUT_) // N_OUT_,
                n_blocks=1, bn=N_OUT_, sigmoid=True)
    return out
